# Initial kernel scaffold; baseline (speedup 1.0000x reference)
import jax
import jax.numpy as jnp
from jax.experimental import pallas as pl


def kernel(x, edge_index, W1, b1, W2, b2):
    raise NotImplementedError("write your pallas kernel here")


# speedup vs baseline: 1.9648x; 1.9648x over previous
import functools
import jax
import jax.numpy as jnp
from jax import lax
from jax.experimental import pallas as pl
from jax.experimental.pallas import tpu as pltpu

NUM_NODES = 100000


def _mm_body(x_ref, w_ref, o_ref):
    o_ref[...] = jnp.dot(x_ref[...], w_ref[...],
                         preferred_element_type=jnp.float32)


def _matmul(x, w, block_rows=4000):
    n, k = x.shape
    m = w.shape[1]
    grid = (n // block_rows,)
    return pl.pallas_call(
        _mm_body,
        grid=grid,
        in_specs=[
            pl.BlockSpec((block_rows, k), lambda i: (i, 0)),
            pl.BlockSpec((k, m), lambda i: (0, 0)),
        ],
        out_specs=pl.BlockSpec((block_rows, m), lambda i: (i, 0)),
        out_shape=jax.ShapeDtypeStruct((n, m), jnp.float32),
    )(x, w)


def kernel(x, edge_index, W1, b1, W2, b2):
    src = edge_index[0].astype(jnp.int32)
    dst = edge_index[1].astype(jnp.int32)

    deg = jnp.zeros((NUM_NODES,), jnp.float32).at[dst].add(1.0) + 1.0
    dinv = lax.rsqrt(deg)

    xw = _matmul(x, W1)
    xs = xw * dinv[:, None]
    agg = jnp.zeros((NUM_NODES, 32), jnp.float32).at[dst].add(xs[src])
    h = jax.nn.relu(dinv[:, None] * agg + dinv[:, None] ** 2 * xw + b1)

    hw = _matmul(h, W2)
    hs = hw * dinv[:, None]
    agg2 = jnp.zeros((NUM_NODES, 2), jnp.float32).at[dst].add(hs[src])
    out = dinv[:, None] * agg2 + dinv[:, None] ** 2 * hw + b2
    return out


# speedup vs baseline: 39.0792x; 19.8897x over previous
import functools
import jax
import jax.numpy as jnp
from jax import lax
from jax.experimental import pallas as pl
from jax.experimental.pallas import tpu as pltpu
from jax.experimental.pallas import tpu_sc as plsc

NN = 100000
EE = 3200000
EPAD = 3211264
NROWS = EPAD // 128
NC, NS = 2, 16
DEG_ACC = 131072
ACC_ROWS = 100096
CH = 8
L1_RPS = NROWS // NS
L2_RPW = NROWS // (NC * NS)
STRIPE = 6256
ZROWS = 1568


def _sc_mesh():
    return plsc.VectorSubcoreMesh(core_axis_name="c", subcore_axis_name="s",
                                  num_cores=NC, num_subcores=NS)


@functools.partial(
    pl.kernel,
    out_type=jax.ShapeDtypeStruct((NC * DEG_ACC,), jnp.float32),
    mesh=_sc_mesh(),
    compiler_params=pltpu.CompilerParams(use_tc_tiling_on_sc=False),
    scratch_types=[
        pltpu.VMEM((CH, 128), jnp.int32),
        pltpu.VMEM((128,), jnp.float32),
        pltpu.VMEM_SHARED((DEG_ACC,), jnp.float32),
        pltpu.SemaphoreType.DMA,
    ])
def _degree_sc(dst_hbm, ones_hbm, zeros_hbm, out_hbm, idx_v, ones_v, acc_sh, sem):
    cid = lax.axis_index("c")
    sid = lax.axis_index("s")
    pltpu.sync_copy(ones_hbm, ones_v)
    pltpu.sync_copy(zeros_hbm, acc_sh.at[pl.ds(sid * 8192, 8192)])
    plsc.subcore_barrier()
    base = (cid * NS + sid) * L2_RPW

    def chunk(t, c):
        r0 = base + t * CH
        pltpu.sync_copy(dst_hbm.at[pl.ds(r0, CH)], idx_v)
        for j in range(CH):
            pltpu.sync_copy(ones_v, acc_sh.at[idx_v.at[j]], add=True)
        return c

    lax.fori_loop(0, L2_RPW // CH, chunk, 0)
    plsc.subcore_barrier()
    pltpu.sync_copy(acc_sh.at[pl.ds(sid * 8192, 8192)],
                    out_hbm.at[pl.ds(cid * DEG_ACC + sid * 8192, 8192)])


def _make_prop(feature_split):
    @functools.partial(
        pl.kernel,
        out_type=jax.ShapeDtypeStruct((NC, NN, 16), jnp.float32),
        mesh=_sc_mesh(),
        compiler_params=pltpu.CompilerParams(use_tc_tiling_on_sc=False),
        scratch_types=[
            pltpu.VMEM((CH, 128), jnp.int32),
            pltpu.VMEM((CH, 128), jnp.int32),
            pltpu.VMEM((CH, 128, 16), jnp.float32),
            pltpu.VMEM_SHARED((ACC_ROWS, 16), jnp.float32),
            pltpu.SemaphoreType.DMA,
        ])
    def prop(src_hbm, dst_hbm, t0_hbm, t1_hbm, zeros_hbm, out_hbm,
             sidx, didx, rows_v, acc_sh, sem):
        cid = lax.axis_index("c")
        sid = lax.axis_index("s")
        for q in range(3):
            pltpu.sync_copy(zeros_hbm,
                            acc_sh.at[pl.ds(sid * STRIPE + q * ZROWS, ZROWS)])
        pltpu.sync_copy(zeros_hbm.at[pl.ds(0, 1552)],
                        acc_sh.at[pl.ds(sid * STRIPE + 3 * ZROWS, 1552)])
        plsc.subcore_barrier()
        if feature_split:
            nrows, base = L1_RPS, sid * L1_RPS
        else:
            nrows, base = L2_RPW, (cid * NS + sid) * L2_RPW

        def chunk(t, c):
            r0 = base + t * CH
            pltpu.sync_copy(src_hbm.at[pl.ds(r0, CH)], sidx)
            pltpu.sync_copy(dst_hbm.at[pl.ds(r0, CH)], didx)

            def gath(tab):
                cps = [pltpu.async_copy(tab.at[sidx.at[j]], rows_v.at[j], sem)
                       for j in range(CH)]
                for cp in cps:
                    cp.wait()

            if feature_split:
                @pl.when(cid == 0)
                def _():
                    gath(t0_hbm)

                @pl.when(cid == 1)
                def _():
                    gath(t1_hbm)
            else:
                gath(t0_hbm)
            for j in range(CH):
                pltpu.sync_copy(rows_v.at[j], acc_sh.at[didx.at[j]], add=True)
            return c

        lax.fori_loop(0, nrows // CH, chunk, 0)
        plsc.subcore_barrier()

        @pl.when(sid < NS - 1)
        def _():
            lo = sid * STRIPE
            pltpu.sync_copy(acc_sh.at[pl.ds(lo, STRIPE)],
                            out_hbm.at[cid, pl.ds(lo, STRIPE)])

        @pl.when(sid == NS - 1)
        def _():
            lo = (NS - 1) * STRIPE
            pltpu.sync_copy(acc_sh.at[pl.ds(lo, NN - lo)],
                            out_hbm.at[cid, pl.ds(lo, NN - lo)])

    return prop


_prop_fsplit = _make_prop(True)
_prop_esplit = _make_prop(False)


_BR = 2000


def _stage_a_body(x_ref, w1_ref, deg_ref, dinv_ref, xs0_ref, xs1_ref, xw_ref):
    dinv = lax.rsqrt(deg_ref[...])
    xw = jnp.dot(x_ref[...], w1_ref[...], preferred_element_type=jnp.float32)
    xs = xw * dinv
    dinv_ref[...] = dinv
    xw_ref[...] = xw
    xs0_ref[...] = xs[:, :16]
    xs1_ref[...] = xs[:, 16:]


def _stage_a(x, w1, deg):
    grid = (NN // _BR,)
    return pl.pallas_call(
        _stage_a_body,
        grid=grid,
        in_specs=[
            pl.BlockSpec((_BR, 16), lambda i: (i, 0)),
            pl.BlockSpec((16, 32), lambda i: (0, 0)),
            pl.BlockSpec((_BR, 1), lambda i: (i, 0)),
        ],
        out_specs=[
            pl.BlockSpec((_BR, 1), lambda i: (i, 0)),
            pl.BlockSpec((_BR, 16), lambda i: (i, 0)),
            pl.BlockSpec((_BR, 16), lambda i: (i, 0)),
            pl.BlockSpec((_BR, 32), lambda i: (i, 0)),
        ],
        out_shape=[
            jax.ShapeDtypeStruct((NN, 1), jnp.float32),
            jax.ShapeDtypeStruct((NN, 16), jnp.float32),
            jax.ShapeDtypeStruct((NN, 16), jnp.float32),
            jax.ShapeDtypeStruct((NN, 32), jnp.float32),
        ],
    )(x, w1, deg)


def _stage_b_body(a0_ref, a1_ref, xw_ref, dinv_ref, b1_ref, w2_ref,
                  hs_ref, hw_ref):
    dinv = dinv_ref[...]
    agg = jnp.concatenate([a0_ref[...], a1_ref[...]], axis=1)
    h = jnp.maximum(dinv * agg + dinv * dinv * xw_ref[...] + b1_ref[...], 0.0)
    hw = jnp.dot(h, w2_ref[...], preferred_element_type=jnp.float32)
    hw_ref[...] = hw
    hs_ref[...] = hw * dinv


def _stage_b(a0, a1, xw, dinv, b1, w2p):
    grid = (NN // _BR,)
    return pl.pallas_call(
        _stage_b_body,
        grid=grid,
        in_specs=[
            pl.BlockSpec((_BR, 16), lambda i: (i, 0)),
            pl.BlockSpec((_BR, 16), lambda i: (i, 0)),
            pl.BlockSpec((_BR, 32), lambda i: (i, 0)),
            pl.BlockSpec((_BR, 1), lambda i: (i, 0)),
            pl.BlockSpec((1, 32), lambda i: (0, 0)),
            pl.BlockSpec((32, 16), lambda i: (0, 0)),
        ],
        out_specs=[
            pl.BlockSpec((_BR, 16), lambda i: (i, 0)),
            pl.BlockSpec((_BR, 16), lambda i: (i, 0)),
        ],
        out_shape=[
            jax.ShapeDtypeStruct((NN, 16), jnp.float32),
            jax.ShapeDtypeStruct((NN, 16), jnp.float32),
        ],
    )(a0, a1, xw, dinv, b1, w2p)


def _stage_c_body(a0_ref, a1_ref, hw_ref, dinv_ref, b2_ref, out_ref):
    dinv = dinv_ref[...]
    agg = a0_ref[...] + a1_ref[...]
    out_ref[...] = dinv * agg + dinv * dinv * hw_ref[...] + b2_ref[...]


def _stage_c(a0, a1, hw, dinv, b2p):
    grid = (NN // _BR,)
    return pl.pallas_call(
        _stage_c_body,
        grid=grid,
        in_specs=[
            pl.BlockSpec((_BR, 16), lambda i: (i, 0)),
            pl.BlockSpec((_BR, 16), lambda i: (i, 0)),
            pl.BlockSpec((_BR, 16), lambda i: (i, 0)),
            pl.BlockSpec((_BR, 1), lambda i: (i, 0)),
            pl.BlockSpec((1, 16), lambda i: (0, 0)),
        ],
        out_specs=pl.BlockSpec((_BR, 16), lambda i: (i, 0)),
        out_shape=jax.ShapeDtypeStruct((NN, 16), jnp.float32),
    )(a0, a1, hw, dinv, b2p)


def kernel(x, edge_index, W1, b1, W2, b2):
    src = edge_index[0].astype(jnp.int32)
    dst = edge_index[1].astype(jnp.int32)

    npad = EPAD - EE
    pad_i = jnp.arange(npad, dtype=jnp.int32)
    src_p = jnp.concatenate([src, pad_i % 128]).reshape(NROWS, 128)
    dst_p = jnp.concatenate([dst, NN + (pad_i % 32)]).reshape(NROWS, 128)

    ones128 = jnp.ones((128,), jnp.float32)
    zeros1d = jnp.zeros((8192,), jnp.float32)
    zeros2d = jnp.zeros((ZROWS, 16), jnp.float32)

    deg_flat = _degree_sc(dst_p, ones128, zeros1d)
    deg_p = deg_flat.reshape(NC, DEG_ACC)
    deg = (deg_p[0, :NN] + deg_p[1, :NN] + 1.0)[:, None]

    dinv, xs0, xs1, xw = _stage_a(x, W1, deg)
    agg1 = _prop_fsplit(src_p, dst_p, xs0, xs1, zeros2d)
    w2p = jnp.pad(W2, ((0, 0), (0, 14)))
    hs, hw = _stage_b(agg1[0], agg1[1], xw, dinv, b1[None, :], w2p)

    agg2 = _prop_esplit(src_p, dst_p, hs, hs, zeros2d)
    b2p = jnp.pad(b2, (0, 14))[None, :]
    out16 = _stage_c(agg2[0], agg2[1], hw, dinv, b2p)
    return out16[:, :2]


# speedup vs baseline: 51.5400x; 1.3189x over previous
import functools
import jax
import jax.numpy as jnp
from jax import lax
from jax.experimental import pallas as pl
from jax.experimental.pallas import tpu as pltpu
from jax.experimental.pallas import tpu_sc as plsc

NN = 100000
EE = 3200000
EPAD = 3211264
NROWS = EPAD // 128
NC, NS = 2, 16
DEG_ACC = 131072
ACC_ROWS = 100096
CH = 8
CHP = 4
L1_RPS = NROWS // NS
L2_RPW = NROWS // (NC * NS)
STRIPE = 6256
ZROWS = 1568


def _sc_mesh():
    return plsc.VectorSubcoreMesh(core_axis_name="c", subcore_axis_name="s",
                                  num_cores=NC, num_subcores=NS)


@functools.partial(
    pl.kernel,
    out_type=jax.ShapeDtypeStruct((NC * DEG_ACC,), jnp.float32),
    mesh=_sc_mesh(),
    compiler_params=pltpu.CompilerParams(use_tc_tiling_on_sc=False),
    scratch_types=[
        pltpu.VMEM((CH, 128), jnp.int32),
        pltpu.VMEM((CH, 128), jnp.int32),
        pltpu.VMEM((128,), jnp.float32),
        pltpu.VMEM_SHARED((DEG_ACC,), jnp.float32),
        pltpu.SemaphoreType.DMA,
        pltpu.SemaphoreType.DMA,
    ])
def _degree_sc(dst_hbm, ones_hbm, zeros_hbm, out_hbm, idx0, idx1, ones_v,
               acc_sh, isem, ssem):
    cid = lax.axis_index("c")
    sid = lax.axis_index("s")
    pltpu.sync_copy(ones_hbm, ones_v)
    pltpu.sync_copy(zeros_hbm, acc_sh.at[pl.ds(sid * 8192, 8192)])
    plsc.subcore_barrier()
    base = (cid * NS + sid) * L2_RPW
    nt = L2_RPW // CH
    bufs = [idx0, idx1]

    def idx_cp(t, b):
        return pltpu.make_async_copy(dst_hbm.at[pl.ds(base + t * CH, CH)],
                                     bufs[b], isem)

    idx_cp(0, 0).start()

    def group(g, c):
        for b in range(2):
            t = g * 2 + b
            idx_cp(t, b).wait()

            @pl.when(t + 1 < nt)
            def _():
                idx_cp(t + 1, 1 - b).start()

            @pl.when(t > 0)
            def _():
                for _j in range(CH):
                    pltpu.make_async_copy(ones_hbm, ones_v, ssem).wait()

            for j in range(CH):
                pltpu.async_copy(ones_v, acc_sh.at[bufs[b].at[j]], ssem,
                                 add=True)
        return c

    lax.fori_loop(0, nt // 2, group, 0)
    for _j in range(CH):
        pltpu.make_async_copy(ones_hbm, ones_v, ssem).wait()
    plsc.subcore_barrier()
    pltpu.sync_copy(acc_sh.at[pl.ds(sid * 8192, 8192)],
                    out_hbm.at[pl.ds(cid * DEG_ACC + sid * 8192, 8192)])


def _make_prop(feature_split):
    @functools.partial(
        pl.kernel,
        out_type=jax.ShapeDtypeStruct((NC, NN, 16), jnp.float32),
        mesh=_sc_mesh(),
        compiler_params=pltpu.CompilerParams(use_tc_tiling_on_sc=False),
        scratch_types=[
            pltpu.VMEM((CHP, 128), jnp.int32),
            pltpu.VMEM((CHP, 128), jnp.int32),
            pltpu.VMEM((CHP, 128), jnp.int32),
            pltpu.VMEM((CHP, 128), jnp.int32),
            pltpu.VMEM((CHP, 128, 16), jnp.float32),
            pltpu.VMEM((CHP, 128, 16), jnp.float32),
            pltpu.VMEM_SHARED((ACC_ROWS, 16), jnp.float32),
            pltpu.SemaphoreType.DMA,
            pltpu.SemaphoreType.DMA,
            pltpu.SemaphoreType.DMA,
        ])
    def prop(src_hbm, dst_hbm, t0_hbm, t1_hbm, zeros_hbm, out_hbm,
             sidx0, sidx1, didx0, didx1, rows0, rows1, acc_sh,
             isem, gsem, ssem):
        cid = lax.axis_index("c")
        sid = lax.axis_index("s")
        for q in range(3):
            pltpu.sync_copy(zeros_hbm,
                            acc_sh.at[pl.ds(sid * STRIPE + q * ZROWS, ZROWS)])
        pltpu.sync_copy(zeros_hbm.at[pl.ds(0, 1552)],
                        acc_sh.at[pl.ds(sid * STRIPE + 3 * ZROWS, 1552)])
        plsc.subcore_barrier()
        if feature_split:
            nrows, base = L1_RPS, sid * L1_RPS
        else:
            nrows, base = L2_RPW, (cid * NS + sid) * L2_RPW
        nt = nrows // CHP
        sbufs, dbufs, rbufs = [sidx0, sidx1], [didx0, didx1], [rows0, rows1]

        def idx_cps(t, b):
            r0 = base + t * CHP
            return (pltpu.make_async_copy(src_hbm.at[pl.ds(r0, CHP)],
                                          sbufs[b], isem),
                    pltpu.make_async_copy(dst_hbm.at[pl.ds(r0, CHP)],
                                          dbufs[b], isem))

        for cp in idx_cps(0, 0):
            cp.start()

        def group(g, c):
            for b in range(2):
                t = g * 2 + b
                for cp in idx_cps(t, b):
                    cp.wait()

                @pl.when(t + 1 < nt)
                def _():
                    for cp in idx_cps(t + 1, 1 - b):
                        cp.start()

                def gath(tab):
                    cps = [pltpu.async_copy(tab.at[sbufs[b].at[j]],
                                            rbufs[b].at[j], gsem)
                           for j in range(CHP)]
                    for cp in cps:
                        cp.wait()

                if feature_split:
                    @pl.when(cid == 0)
                    def _():
                        gath(t0_hbm)

                    @pl.when(cid == 1)
                    def _():
                        gath(t1_hbm)
                else:
                    gath(t0_hbm)

                @pl.when(t > 0)
                def _():
                    for j in range(CHP):
                        pltpu.make_async_copy(t0_hbm.at[pl.ds(0, 128)],
                                              rbufs[1 - b].at[j], ssem).wait()

                for j in range(CHP):
                    pltpu.async_copy(rbufs[b].at[j], acc_sh.at[dbufs[b].at[j]],
                                     ssem, add=True)
            return c

        lax.fori_loop(0, nt // 2, group, 0)
        for j in range(CHP):
            pltpu.make_async_copy(t0_hbm.at[pl.ds(0, 128)],
                                  rows1.at[j], ssem).wait()
        plsc.subcore_barrier()

        @pl.when(sid < NS - 1)
        def _():
            lo = sid * STRIPE
            pltpu.sync_copy(acc_sh.at[pl.ds(lo, STRIPE)],
                            out_hbm.at[cid, pl.ds(lo, STRIPE)])

        @pl.when(sid == NS - 1)
        def _():
            lo = (NS - 1) * STRIPE
            pltpu.sync_copy(acc_sh.at[pl.ds(lo, NN - lo)],
                            out_hbm.at[cid, pl.ds(lo, NN - lo)])

    return prop


_prop_fsplit = _make_prop(True)
_prop_esplit = _make_prop(False)


_BR = 2000


def _stage_a_body(x_ref, w1_ref, deg_ref, dinv_ref, xs0_ref, xs1_ref, xw_ref):
    dinv = lax.rsqrt(deg_ref[...])
    xw = jnp.dot(x_ref[...], w1_ref[...], preferred_element_type=jnp.float32)
    xs = xw * dinv
    dinv_ref[...] = dinv
    xw_ref[...] = xw
    xs0_ref[...] = xs[:, :16]
    xs1_ref[...] = xs[:, 16:]


def _stage_a(x, w1, deg):
    grid = (NN // _BR,)
    return pl.pallas_call(
        _stage_a_body,
        grid=grid,
        in_specs=[
            pl.BlockSpec((_BR, 16), lambda i: (i, 0)),
            pl.BlockSpec((16, 32), lambda i: (0, 0)),
            pl.BlockSpec((_BR, 1), lambda i: (i, 0)),
        ],
        out_specs=[
            pl.BlockSpec((_BR, 1), lambda i: (i, 0)),
            pl.BlockSpec((_BR, 16), lambda i: (i, 0)),
            pl.BlockSpec((_BR, 16), lambda i: (i, 0)),
            pl.BlockSpec((_BR, 32), lambda i: (i, 0)),
        ],
        out_shape=[
            jax.ShapeDtypeStruct((NN, 1), jnp.float32),
            jax.ShapeDtypeStruct((NN, 16), jnp.float32),
            jax.ShapeDtypeStruct((NN, 16), jnp.float32),
            jax.ShapeDtypeStruct((NN, 32), jnp.float32),
        ],
    )(x, w1, deg)


def _stage_b_body(a0_ref, a1_ref, xw_ref, dinv_ref, b1_ref, w2_ref,
                  hs_ref, hw_ref):
    dinv = dinv_ref[...]
    agg = jnp.concatenate([a0_ref[...], a1_ref[...]], axis=1)
    h = jnp.maximum(dinv * agg + dinv * dinv * xw_ref[...] + b1_ref[...], 0.0)
    hw = jnp.dot(h, w2_ref[...], preferred_element_type=jnp.float32)
    hw_ref[...] = hw
    hs_ref[...] = hw * dinv


def _stage_b(a0, a1, xw, dinv, b1, w2p):
    grid = (NN // _BR,)
    return pl.pallas_call(
        _stage_b_body,
        grid=grid,
        in_specs=[
            pl.BlockSpec((_BR, 16), lambda i: (i, 0)),
            pl.BlockSpec((_BR, 16), lambda i: (i, 0)),
            pl.BlockSpec((_BR, 32), lambda i: (i, 0)),
            pl.BlockSpec((_BR, 1), lambda i: (i, 0)),
            pl.BlockSpec((1, 32), lambda i: (0, 0)),
            pl.BlockSpec((32, 16), lambda i: (0, 0)),
        ],
        out_specs=[
            pl.BlockSpec((_BR, 16), lambda i: (i, 0)),
            pl.BlockSpec((_BR, 16), lambda i: (i, 0)),
        ],
        out_shape=[
            jax.ShapeDtypeStruct((NN, 16), jnp.float32),
            jax.ShapeDtypeStruct((NN, 16), jnp.float32),
        ],
    )(a0, a1, xw, dinv, b1, w2p)


def _stage_c_body(a0_ref, a1_ref, hw_ref, dinv_ref, b2_ref, out_ref):
    dinv = dinv_ref[...]
    agg = a0_ref[...] + a1_ref[...]
    out_ref[...] = dinv * agg + dinv * dinv * hw_ref[...] + b2_ref[...]


def _stage_c(a0, a1, hw, dinv, b2p):
    grid = (NN // _BR,)
    return pl.pallas_call(
        _stage_c_body,
        grid=grid,
        in_specs=[
            pl.BlockSpec((_BR, 16), lambda i: (i, 0)),
            pl.BlockSpec((_BR, 16), lambda i: (i, 0)),
            pl.BlockSpec((_BR, 16), lambda i: (i, 0)),
            pl.BlockSpec((_BR, 1), lambda i: (i, 0)),
            pl.BlockSpec((1, 16), lambda i: (0, 0)),
        ],
        out_specs=pl.BlockSpec((_BR, 16), lambda i: (i, 0)),
        out_shape=jax.ShapeDtypeStruct((NN, 16), jnp.float32),
    )(a0, a1, hw, dinv, b2p)


def kernel(x, edge_index, W1, b1, W2, b2):
    src = edge_index[0].astype(jnp.int32)
    dst = edge_index[1].astype(jnp.int32)

    npad = EPAD - EE
    pad_i = jnp.arange(npad, dtype=jnp.int32)
    src_p = jnp.concatenate([src, pad_i % 128]).reshape(NROWS, 128)
    dst_p = jnp.concatenate([dst, NN + (pad_i % 32)]).reshape(NROWS, 128)

    ones128 = jnp.ones((128,), jnp.float32)
    zeros1d = jnp.zeros((8192,), jnp.float32)
    zeros2d = jnp.zeros((ZROWS, 16), jnp.float32)

    deg_flat = _degree_sc(dst_p, ones128, zeros1d)
    deg_p = deg_flat.reshape(NC, DEG_ACC)
    deg = (deg_p[0, :NN] + deg_p[1, :NN] + 1.0)[:, None]

    dinv, xs0, xs1, xw = _stage_a(x, W1, deg)
    agg1 = _prop_fsplit(src_p, dst_p, xs0, xs1, zeros2d)
    w2p = jnp.pad(W2, ((0, 0), (0, 14)))
    hs, hw = _stage_b(agg1[0], agg1[1], xw, dinv, b1[None, :], w2p)

    agg2 = _prop_esplit(src_p, dst_p, hs, hs, zeros2d)
    b2p = jnp.pad(b2, (0, 14))[None, :]
    out16 = _stage_c(agg2[0], agg2[1], hw, dinv, b2p)
    return out16[:, :2]


# speedup vs baseline: 56.5232x; 1.0967x over previous
import functools
import jax
import jax.numpy as jnp
from jax import lax
from jax.experimental import pallas as pl
from jax.experimental.pallas import tpu as pltpu
from jax.experimental.pallas import tpu_sc as plsc

NN = 100000
EE = 3200000
EPAD = 3211264
NROWS = EPAD // 128
NC, NS = 2, 16
DEG_ACC = 131072
ACC_ROWS = 100096
CH = 8
CHP = 4
L1_RPS = NROWS // NS
L2_RPW = NROWS // (NC * NS)
STRIPE = 6256
ZROWS = 1568


def _sc_mesh():
    return plsc.VectorSubcoreMesh(core_axis_name="c", subcore_axis_name="s",
                                  num_cores=NC, num_subcores=NS)


@functools.partial(
    pl.kernel,
    out_type=jax.ShapeDtypeStruct((NC * DEG_ACC,), jnp.float32),
    mesh=_sc_mesh(),
    compiler_params=pltpu.CompilerParams(use_tc_tiling_on_sc=False),
    scratch_types=[
        pltpu.VMEM((CH, 128), jnp.int32),
        pltpu.VMEM((CH, 128), jnp.int32),
        pltpu.VMEM((128,), jnp.float32),
        pltpu.VMEM_SHARED((DEG_ACC,), jnp.float32),
        pltpu.SemaphoreType.DMA,
        pltpu.SemaphoreType.DMA,
    ])
def _degree_sc(dst_hbm, ones_hbm, zeros_hbm, out_hbm, idx0, idx1, ones_v,
               acc_sh, isem, ssem):
    cid = lax.axis_index("c")
    sid = lax.axis_index("s")
    pltpu.sync_copy(ones_hbm, ones_v)
    pltpu.sync_copy(zeros_hbm, acc_sh.at[pl.ds(sid * 8192, 8192)])
    plsc.subcore_barrier()
    base = (cid * NS + sid) * L2_RPW
    nt = L2_RPW // CH
    bufs = [idx0, idx1]

    def idx_cp(t, b):
        return pltpu.make_async_copy(dst_hbm.at[pl.ds(base + t * CH, CH)],
                                     bufs[b], isem)

    idx_cp(0, 0).start()

    def group(g, c):
        for b in range(2):
            t = g * 2 + b
            idx_cp(t, b).wait()

            @pl.when(t + 1 < nt)
            def _():
                idx_cp(t + 1, 1 - b).start()

            @pl.when(t > 0)
            def _():
                for _j in range(CH):
                    pltpu.make_async_copy(ones_hbm, ones_v, ssem).wait()

            for j in range(CH):
                pltpu.async_copy(ones_v, acc_sh.at[bufs[b].at[j]], ssem,
                                 add=True)
        return c

    lax.fori_loop(0, nt // 2, group, 0)
    for _j in range(CH):
        pltpu.make_async_copy(ones_hbm, ones_v, ssem).wait()
    plsc.subcore_barrier()
    pltpu.sync_copy(acc_sh.at[pl.ds(sid * 8192, 8192)],
                    out_hbm.at[pl.ds(cid * DEG_ACC + sid * 8192, 8192)])


def _make_prop(feature_split):
    @functools.partial(
        pl.kernel,
        out_type=jax.ShapeDtypeStruct((NC, NN, 16), jnp.float32),
        mesh=_sc_mesh(),
        compiler_params=pltpu.CompilerParams(use_tc_tiling_on_sc=False),
        scratch_types=[
            pltpu.VMEM((CHP, 128), jnp.int32),
            pltpu.VMEM((CHP, 128), jnp.int32),
            pltpu.VMEM((CHP, 128), jnp.int32),
            pltpu.VMEM((CHP, 128), jnp.int32),
            pltpu.VMEM((CHP, 128, 16), jnp.float32),
            pltpu.VMEM((CHP, 128, 16), jnp.float32),
            pltpu.VMEM_SHARED((ACC_ROWS, 16), jnp.float32),
            pltpu.SemaphoreType.DMA,
            pltpu.SemaphoreType.DMA,
            pltpu.SemaphoreType.DMA,
        ])
    def prop(src_hbm, dst_hbm, t0_hbm, t1_hbm, zeros_hbm, out_hbm,
             sidx0, sidx1, didx0, didx1, rows0, rows1, acc_sh,
             isem, gsem, ssem):
        cid = lax.axis_index("c")
        sid = lax.axis_index("s")
        for q in range(3):
            pltpu.sync_copy(zeros_hbm,
                            acc_sh.at[pl.ds(sid * STRIPE + q * ZROWS, ZROWS)])
        pltpu.sync_copy(zeros_hbm.at[pl.ds(0, 1552)],
                        acc_sh.at[pl.ds(sid * STRIPE + 3 * ZROWS, 1552)])
        plsc.subcore_barrier()
        if feature_split:
            nrows, base = L1_RPS, sid * L1_RPS
        else:
            nrows, base = L2_RPW, (cid * NS + sid) * L2_RPW
        nt = nrows // CHP
        sbufs, dbufs, rbufs = [sidx0, sidx1], [didx0, didx1], [rows0, rows1]

        def idx_cps(t, b):
            r0 = base + t * CHP
            return (pltpu.make_async_copy(src_hbm.at[pl.ds(r0, CHP)],
                                          sbufs[b], isem),
                    pltpu.make_async_copy(dst_hbm.at[pl.ds(r0, CHP)],
                                          dbufs[b], isem))

        for cp in idx_cps(0, 0):
            cp.start()

        def group(g, c):
            for b in range(2):
                t = g * 2 + b
                for cp in idx_cps(t, b):
                    cp.wait()

                @pl.when(t + 1 < nt)
                def _():
                    for cp in idx_cps(t + 1, 1 - b):
                        cp.start()

                def gath(tab):
                    cps = [pltpu.async_copy(tab.at[sbufs[b].at[j]],
                                            rbufs[b].at[j], gsem)
                           for j in range(CHP)]
                    for cp in cps:
                        cp.wait()

                if feature_split:
                    @pl.when(cid == 0)
                    def _():
                        gath(t0_hbm)

                    @pl.when(cid == 1)
                    def _():
                        gath(t1_hbm)
                else:
                    gath(t0_hbm)

                @pl.when(t > 0)
                def _():
                    for j in range(CHP):
                        pltpu.make_async_copy(t0_hbm.at[pl.ds(0, 128)],
                                              rbufs[1 - b].at[j], ssem).wait()

                for j in range(CHP):
                    pltpu.async_copy(rbufs[b].at[j], acc_sh.at[dbufs[b].at[j]],
                                     ssem, add=True)
            return c

        lax.fori_loop(0, nt // 2, group, 0)
        for j in range(CHP):
            pltpu.make_async_copy(t0_hbm.at[pl.ds(0, 128)],
                                  rows1.at[j], ssem).wait()
        plsc.subcore_barrier()

        @pl.when(sid < NS - 1)
        def _():
            lo = sid * STRIPE
            pltpu.sync_copy(acc_sh.at[pl.ds(lo, STRIPE)],
                            out_hbm.at[cid, pl.ds(lo, STRIPE)])

        @pl.when(sid == NS - 1)
        def _():
            lo = (NS - 1) * STRIPE
            pltpu.sync_copy(acc_sh.at[pl.ds(lo, NN - lo)],
                            out_hbm.at[cid, pl.ds(lo, NN - lo)])

    return prop


_prop_fsplit = _make_prop(True)
_prop_esplit = _make_prop(False)


_BR = 5000


def _stage_a_body(x_ref, w1_ref, dinv_ref, xs0_ref, xs1_ref, xw_ref):
    dinv16 = dinv_ref[...]
    xw = jnp.dot(x_ref[...], w1_ref[...], preferred_element_type=jnp.float32)
    xw_ref[...] = xw
    xs0_ref[...] = xw[:, :16] * dinv16
    xs1_ref[...] = xw[:, 16:] * dinv16


def _stage_a(x, w1, dinv16):
    grid = (NN // _BR,)
    return pl.pallas_call(
        _stage_a_body,
        grid=grid,
        in_specs=[
            pl.BlockSpec((_BR, 16), lambda i: (i, 0)),
            pl.BlockSpec((16, 32), lambda i: (0, 0)),
            pl.BlockSpec((_BR, 16), lambda i: (i, 0)),
        ],
        out_specs=[
            pl.BlockSpec((_BR, 16), lambda i: (i, 0)),
            pl.BlockSpec((_BR, 16), lambda i: (i, 0)),
            pl.BlockSpec((_BR, 32), lambda i: (i, 0)),
        ],
        out_shape=[
            jax.ShapeDtypeStruct((NN, 16), jnp.float32),
            jax.ShapeDtypeStruct((NN, 16), jnp.float32),
            jax.ShapeDtypeStruct((NN, 32), jnp.float32),
        ],
    )(x, w1, dinv16)


def _stage_b_body(a0_ref, a1_ref, xw_ref, dinv_ref, b1_ref, w2_ref,
                  hs_ref, hw_ref):
    dinv16 = dinv_ref[...]
    dinv2 = dinv16 * dinv16
    xw = xw_ref[...]
    pre = jnp.concatenate(
        [a0_ref[0] * dinv16 + xw[:, :16] * dinv2,
         a1_ref[0] * dinv16 + xw[:, 16:] * dinv2], axis=1)
    h = jnp.maximum(pre + b1_ref[...], 0.0)
    hw = jnp.dot(h, w2_ref[...], preferred_element_type=jnp.float32)
    hw_ref[...] = hw
    hs_ref[...] = hw * dinv16


def _stage_b(agg, xw, dinv16, b1, w2p):
    grid = (NN // _BR,)
    return pl.pallas_call(
        _stage_b_body,
        grid=grid,
        in_specs=[
            pl.BlockSpec((1, _BR, 16), lambda i: (0, i, 0)),
            pl.BlockSpec((1, _BR, 16), lambda i: (1, i, 0)),
            pl.BlockSpec((_BR, 32), lambda i: (i, 0)),
            pl.BlockSpec((_BR, 16), lambda i: (i, 0)),
            pl.BlockSpec((1, 32), lambda i: (0, 0)),
            pl.BlockSpec((32, 16), lambda i: (0, 0)),
        ],
        out_specs=[
            pl.BlockSpec((_BR, 16), lambda i: (i, 0)),
            pl.BlockSpec((_BR, 16), lambda i: (i, 0)),
        ],
        out_shape=[
            jax.ShapeDtypeStruct((NN, 16), jnp.float32),
            jax.ShapeDtypeStruct((NN, 16), jnp.float32),
        ],
    )(agg, agg, xw, dinv16, b1, w2p)


def _stage_c_body(a0_ref, a1_ref, hw_ref, dinv_ref, b2_ref, out_ref):
    dinv16 = dinv_ref[...]
    agg = a0_ref[0] + a1_ref[0]
    out_ref[...] = dinv16 * agg + dinv16 * dinv16 * hw_ref[...] + b2_ref[...]


def _stage_c(agg2, hw, dinv16, b2p):
    grid = (NN // _BR,)
    return pl.pallas_call(
        _stage_c_body,
        grid=grid,
        in_specs=[
            pl.BlockSpec((1, _BR, 16), lambda i: (0, i, 0)),
            pl.BlockSpec((1, _BR, 16), lambda i: (1, i, 0)),
            pl.BlockSpec((_BR, 16), lambda i: (i, 0)),
            pl.BlockSpec((_BR, 16), lambda i: (i, 0)),
            pl.BlockSpec((1, 16), lambda i: (0, 0)),
        ],
        out_specs=pl.BlockSpec((_BR, 16), lambda i: (i, 0)),
        out_shape=jax.ShapeDtypeStruct((NN, 16), jnp.float32),
    )(agg2, agg2, hw, dinv16, b2p)


def kernel(x, edge_index, W1, b1, W2, b2):
    src = edge_index[0].astype(jnp.int32)
    dst = edge_index[1].astype(jnp.int32)

    npad = EPAD - EE
    pad_i = jnp.arange(npad, dtype=jnp.int32)
    src_p = jnp.concatenate([src, pad_i % 128]).reshape(NROWS, 128)
    dst_p = jnp.concatenate([dst, NN + (pad_i % 32)]).reshape(NROWS, 128)

    ones128 = jnp.ones((128,), jnp.float32)
    zeros1d = jnp.zeros((8192,), jnp.float32)
    zeros2d = jnp.zeros((ZROWS, 16), jnp.float32)

    deg_flat = _degree_sc(dst_p, ones128, zeros1d)
    deg1d = deg_flat[:NN] + deg_flat[DEG_ACC:DEG_ACC + NN] + 1.0
    dinv16 = jnp.broadcast_to(lax.rsqrt(deg1d)[:, None], (NN, 16))

    xs0, xs1, xw = _stage_a(x, W1, dinv16)
    agg1 = _prop_fsplit(src_p, dst_p, xs0, xs1, zeros2d)
    w2p = jnp.pad(W2, ((0, 0), (0, 14)))
    hs, hw = _stage_b(agg1, xw, dinv16, b1[None, :], w2p)

    agg2 = _prop_esplit(src_p, dst_p, hs, hs, zeros2d)
    b2p = jnp.pad(b2, (0, 14))[None, :]
    out16 = _stage_c(agg2, hw, dinv16, b2p)
    return out16[:, :2]


# speedup vs baseline: 75.5070x; 1.3359x over previous
import functools
import jax
import jax.numpy as jnp
from jax import lax
from jax.experimental import pallas as pl
from jax.experimental.pallas import tpu as pltpu
from jax.experimental.pallas import tpu_sc as plsc

NN = 100000
EE = 3200000
EPAD = 3211264
NROWS = EPAD // 128
NC, NS = 2, 16
DEG_ACC = 131072
ACC_ROWS = 100096
CH = 8
CHP = 4
L1_RPS = NROWS // NS
L2_RPW = NROWS // (NC * NS)
STRIPE = 6256
ZROWS = 1568


def _sc_mesh():
    return plsc.VectorSubcoreMesh(core_axis_name="c", subcore_axis_name="s",
                                  num_cores=NC, num_subcores=NS)


@functools.partial(
    pl.kernel,
    out_type=jax.ShapeDtypeStruct((NC * DEG_ACC,), jnp.float32),
    mesh=_sc_mesh(),
    compiler_params=pltpu.CompilerParams(use_tc_tiling_on_sc=False),
    scratch_types=[
        pltpu.VMEM((CH, 128), jnp.int32),
        pltpu.VMEM((CH, 128), jnp.int32),
        pltpu.VMEM((128,), jnp.float32),
        pltpu.VMEM_SHARED((DEG_ACC,), jnp.float32),
        pltpu.SemaphoreType.DMA,
        pltpu.SemaphoreType.DMA,
    ])
def _degree_sc(dst_hbm, ones_hbm, zeros_hbm, out_hbm, idx0, idx1, ones_v,
               acc_sh, isem, ssem):
    cid = lax.axis_index("c")
    sid = lax.axis_index("s")
    pltpu.sync_copy(ones_hbm, ones_v)
    pltpu.sync_copy(zeros_hbm, acc_sh.at[pl.ds(sid * 8192, 8192)])
    plsc.subcore_barrier()
    base = (cid * NS + sid) * L2_RPW
    nt = L2_RPW // CH
    bufs = [idx0, idx1]

    def idx_cp(t, b):
        return pltpu.make_async_copy(dst_hbm.at[pl.ds(base + t * CH, CH)],
                                     bufs[b], isem)

    idx_cp(0, 0).start()

    def group(g, c):
        for b in range(2):
            t = g * 2 + b
            idx_cp(t, b).wait()

            @pl.when(t + 1 < nt)
            def _():
                idx_cp(t + 1, 1 - b).start()

            @pl.when(t > 0)
            def _():
                for _j in range(CH):
                    pltpu.make_async_copy(ones_hbm, ones_v, ssem).wait()

            for j in range(CH):
                pltpu.async_copy(ones_v, acc_sh.at[bufs[b].at[j]], ssem,
                                 add=True)
        return c

    lax.fori_loop(0, nt // 2, group, 0)
    for _j in range(CH):
        pltpu.make_async_copy(ones_hbm, ones_v, ssem).wait()
    plsc.subcore_barrier()
    pltpu.sync_copy(acc_sh.at[pl.ds(sid * 8192, 8192)],
                    out_hbm.at[pl.ds(cid * DEG_ACC + sid * 8192, 8192)])


def _make_prop(feature_split):
    @functools.partial(
        pl.kernel,
        out_type=jax.ShapeDtypeStruct((NC, NN, 16), jnp.float32),
        mesh=_sc_mesh(),
        compiler_params=pltpu.CompilerParams(use_tc_tiling_on_sc=False),
        scratch_types=[
            pltpu.VMEM((CHP, 128), jnp.int32),
            pltpu.VMEM((CHP, 128), jnp.int32),
            pltpu.VMEM((CHP, 128), jnp.int32),
            pltpu.VMEM((CHP, 128), jnp.int32),
            pltpu.VMEM((CHP, 128, 16), jnp.float32),
            pltpu.VMEM((CHP, 128, 16), jnp.float32),
            pltpu.VMEM_SHARED((ACC_ROWS, 16), jnp.float32),
            pltpu.SemaphoreType.DMA,
            pltpu.SemaphoreType.DMA,
            pltpu.SemaphoreType.DMA,
        ])
    def prop(src_hbm, dst_hbm, t0_hbm, t1_hbm, zeros_hbm, out_hbm,
             sidx0, sidx1, didx0, didx1, rows0, rows1, acc_sh,
             isem, gsem, ssem):
        cid = lax.axis_index("c")
        sid = lax.axis_index("s")
        for q in range(3):
            pltpu.sync_copy(zeros_hbm,
                            acc_sh.at[pl.ds(sid * STRIPE + q * ZROWS, ZROWS)])
        pltpu.sync_copy(zeros_hbm.at[pl.ds(0, 1552)],
                        acc_sh.at[pl.ds(sid * STRIPE + 3 * ZROWS, 1552)])
        plsc.subcore_barrier()
        if feature_split:
            nrows, base = L1_RPS, sid * L1_RPS
        else:
            nrows, base = L2_RPW, (cid * NS + sid) * L2_RPW
        nt = nrows // CHP
        sbufs, dbufs, rbufs = [sidx0, sidx1], [didx0, didx1], [rows0, rows1]

        def idx_cps(t, b):
            r0 = base + t * CHP
            return (pltpu.make_async_copy(src_hbm.at[pl.ds(r0, CHP)],
                                          sbufs[b], isem),
                    pltpu.make_async_copy(dst_hbm.at[pl.ds(r0, CHP)],
                                          dbufs[b], isem))

        for cp in idx_cps(0, 0):
            cp.start()

        def group(g, c):
            for b in range(2):
                t = g * 2 + b
                for cp in idx_cps(t, b):
                    cp.wait()

                @pl.when(t + 1 < nt)
                def _():
                    for cp in idx_cps(t + 1, 1 - b):
                        cp.start()

                def gath(tab):
                    cps = [pltpu.async_copy(tab.at[sbufs[b].at[j]],
                                            rbufs[b].at[j], gsem)
                           for j in range(CHP)]
                    for cp in cps:
                        cp.wait()

                if feature_split:
                    @pl.when(cid == 0)
                    def _():
                        gath(t0_hbm)

                    @pl.when(cid == 1)
                    def _():
                        gath(t1_hbm)
                else:
                    gath(t0_hbm)

                @pl.when(t > 0)
                def _():
                    for j in range(CHP):
                        pltpu.make_async_copy(t0_hbm.at[pl.ds(0, 128)],
                                              rbufs[1 - b].at[j], ssem).wait()

                for j in range(CHP):
                    pltpu.async_copy(rbufs[b].at[j], acc_sh.at[dbufs[b].at[j]],
                                     ssem, add=True)
            return c

        lax.fori_loop(0, nt // 2, group, 0)
        for j in range(CHP):
            pltpu.make_async_copy(t0_hbm.at[pl.ds(0, 128)],
                                  rows1.at[j], ssem).wait()
        plsc.subcore_barrier()

        @pl.when(sid < NS - 1)
        def _():
            lo = sid * STRIPE
            pltpu.sync_copy(acc_sh.at[pl.ds(lo, STRIPE)],
                            out_hbm.at[cid, pl.ds(lo, STRIPE)])

        @pl.when(sid == NS - 1)
        def _():
            lo = (NS - 1) * STRIPE
            pltpu.sync_copy(acc_sh.at[pl.ds(lo, NN - lo)],
                            out_hbm.at[cid, pl.ds(lo, NN - lo)])

    return prop


_prop_fsplit = _make_prop(True)
_prop_esplit = _make_prop(False)


GW, RB = 10, 1250
_W3 = (GW, RB, 128)


def _stage_a_body(x_ref, w1a_ref, w1b_ref, dinv_ref, xs0_ref, xs1_ref,
                  xw0_ref, xw1_ref):
    x = x_ref[0]
    dinv = dinv_ref[0]
    xw0 = jnp.dot(x, w1a_ref[...], preferred_element_type=jnp.float32)
    xw1 = jnp.dot(x, w1b_ref[...], preferred_element_type=jnp.float32)
    xw0_ref[0] = xw0
    xw1_ref[0] = xw1
    xs0_ref[0] = xw0 * dinv
    xs1_ref[0] = xw1 * dinv


_BS = pl.BlockSpec((1, RB, 128), lambda i: (i, 0, 0))
_WS = pl.BlockSpec((128, 128), lambda i: (0, 0))
_VS = pl.BlockSpec((1, 128), lambda i: (0, 0))
_AS0 = pl.BlockSpec((1, 1, RB, 128), lambda i: (0, i, 0, 0))
_AS1 = pl.BlockSpec((1, 1, RB, 128), lambda i: (1, i, 0, 0))
_OW = jax.ShapeDtypeStruct(_W3, jnp.float32)


def _stage_a(xw_wide, w1a, w1b, dinvw):
    return pl.pallas_call(
        _stage_a_body,
        grid=(GW,),
        in_specs=[_BS, _WS, _WS, _BS],
        out_specs=[_BS, _BS, _BS, _BS],
        out_shape=[_OW, _OW, _OW, _OW],
    )(xw_wide, w1a, w1b, dinvw)


def _stage_b_body(a0_ref, a1_ref, xw0_ref, xw1_ref, dinv_ref, b1lo_ref,
                  b1hi_ref, w2a_ref, w2b_ref, hs_ref, hw_ref):
    dinv = dinv_ref[0]
    d2 = dinv * dinv
    h0 = jnp.maximum(a0_ref[0, 0] * dinv + xw0_ref[0] * d2 + b1lo_ref[...], 0.)
    h1 = jnp.maximum(a1_ref[0, 0] * dinv + xw1_ref[0] * d2 + b1hi_ref[...], 0.)
    hw = (jnp.dot(h0, w2a_ref[...], preferred_element_type=jnp.float32)
          + jnp.dot(h1, w2b_ref[...], preferred_element_type=jnp.float32))
    hw_ref[0] = hw
    hs_ref[0] = hw * dinv


def _stage_b(aggw, xw0, xw1, dinvw, b1lo, b1hi, w2a, w2b):
    return pl.pallas_call(
        _stage_b_body,
        grid=(GW,),
        in_specs=[_AS0, _AS1, _BS, _BS, _BS, _VS, _VS, _WS, _WS],
        out_specs=[_BS, _BS],
        out_shape=[_OW, _OW],
    )(aggw, aggw, xw0, xw1, dinvw, b1lo, b1hi, w2a, w2b)


def _stage_c_body(a0_ref, a1_ref, hw_ref, dinv_ref, b2_ref, out_ref):
    dinv = dinv_ref[0]
    agg = a0_ref[0, 0] + a1_ref[0, 0]
    out_ref[0] = dinv * agg + dinv * dinv * hw_ref[0] + b2_ref[...]


def _stage_c(agg2w, hw, dinvw, b2w):
    return pl.pallas_call(
        _stage_c_body,
        grid=(GW,),
        in_specs=[_AS0, _AS1, _BS, _BS, _VS],
        out_specs=_BS,
        out_shape=_OW,
    )(agg2w, agg2w, hw, dinvw, b2w)


def kernel(x, edge_index, W1, b1, W2, b2):
    src = edge_index[0].astype(jnp.int32)
    dst = edge_index[1].astype(jnp.int32)

    npad = EPAD - EE
    pad_i = jnp.arange(npad, dtype=jnp.int32)
    src_p = jnp.concatenate([src, pad_i % 128]).reshape(NROWS, 128)
    dst_p = jnp.concatenate([dst, NN + (pad_i % 32)]).reshape(NROWS, 128)

    ones128 = jnp.ones((128,), jnp.float32)
    zeros1d = jnp.zeros((8192,), jnp.float32)
    zeros2d = jnp.zeros((ZROWS, 16), jnp.float32)

    eye8 = jnp.eye(8, dtype=jnp.float32)
    w1a = jnp.kron(eye8, W1[:, :16])
    w1b = jnp.kron(eye8, W1[:, 16:])
    w2p = jnp.pad(W2, ((0, 0), (0, 14)))
    w2a = jnp.kron(eye8, w2p[:16, :])
    w2b = jnp.kron(eye8, w2p[16:, :])
    b1lo = jnp.tile(b1[:16], 8)[None, :]
    b1hi = jnp.tile(b1[16:], 8)[None, :]
    b2w = jnp.tile(jnp.pad(b2, (0, 14)), 8)[None, :]

    deg_flat = _degree_sc(dst_p, ones128, zeros1d)
    deg1d = deg_flat[:NN] + deg_flat[DEG_ACC:DEG_ACC + NN] + 1.0
    dinvw = jnp.broadcast_to(lax.rsqrt(deg1d)[:, None], (NN, 16)).reshape(_W3)

    x_wide = x.reshape(_W3)
    xs0w, xs1w, xw0w, xw1w = _stage_a(x_wide, w1a, w1b, dinvw)
    agg1 = _prop_fsplit(src_p, dst_p, xs0w.reshape(NN, 16),
                        xs1w.reshape(NN, 16), zeros2d)
    hsw, hww = _stage_b(agg1.reshape((NC,) + _W3), xw0w, xw1w, dinvw,
                        b1lo, b1hi, w2a, w2b)

    hs = hsw.reshape(NN, 16)
    agg2 = _prop_esplit(src_p, dst_p, hs, hs, zeros2d)
    out16w = _stage_c(agg2.reshape((NC,) + _W3), hww, dinvw, b2w)
    return out16w.reshape(NN, 16)[:, :2]


# speedup vs baseline: 92.0914x; 1.2196x over previous
import functools
import jax
import jax.numpy as jnp
from jax import lax
from jax.experimental import pallas as pl
from jax.experimental.pallas import tpu as pltpu
from jax.experimental.pallas import tpu_sc as plsc

NN = 100000
EE = 3200000
EPAD = 3244032
NROWS = EPAD // 128
NC, NS = 2, 16
DEG_ACC = 131072
ACC_ROWS = 100096
CH = 4
CHP = 4
L1_RPS = NROWS // NS
L2_RPW = NROWS // (NC * NS)
STRIPE = 6256
ZROWS = 1568


def _sc_mesh():
    return plsc.VectorSubcoreMesh(core_axis_name="c", subcore_axis_name="s",
                                  num_cores=NC, num_subcores=NS)


@functools.partial(
    pl.kernel,
    out_type=jax.ShapeDtypeStruct((NC * DEG_ACC,), jnp.float32),
    mesh=_sc_mesh(),
    compiler_params=pltpu.CompilerParams(use_tc_tiling_on_sc=False),
    scratch_types=[
        pltpu.VMEM((CH, 128), jnp.int32),
        pltpu.VMEM((CH, 128), jnp.int32),
        pltpu.VMEM((128,), jnp.float32),
        pltpu.VMEM_SHARED((DEG_ACC,), jnp.float32),
        pltpu.SemaphoreType.DMA,
        pltpu.SemaphoreType.DMA,
    ])
def _degree_sc(dst_hbm, ones_hbm, zeros_hbm, out_hbm, idx0, idx1, ones_v,
               acc_sh, isem, ssem):
    cid = lax.axis_index("c")
    sid = lax.axis_index("s")
    pltpu.sync_copy(ones_hbm, ones_v)
    pltpu.sync_copy(zeros_hbm, acc_sh.at[pl.ds(sid * 8192, 8192)])
    plsc.subcore_barrier()
    base = (cid * NS + sid) * L2_RPW
    nt = L2_RPW // CH
    bufs = [idx0, idx1]

    def idx_cp(t, b):
        return pltpu.make_async_copy(dst_hbm.at[pl.ds(base + t * CH, CH)],
                                     bufs[b], isem)

    idx_cp(0, 0).start()

    def group(g, c):
        for b in range(2):
            t = g * 2 + b
            idx_cp(t, b).wait()

            @pl.when(t > 0)
            def _():
                for _j in range(CH):
                    pltpu.make_async_copy(ones_hbm, ones_v, ssem).wait()

            @pl.when(t + 1 < nt)
            def _():
                idx_cp(t + 1, 1 - b).start()

            for j in range(CH):
                pltpu.async_copy(ones_v, acc_sh.at[bufs[b].at[j]], ssem,
                                 add=True)
        return c

    lax.fori_loop(0, nt // 2, group, 0)
    for _j in range(CH):
        pltpu.make_async_copy(ones_hbm, ones_v, ssem).wait()
    plsc.subcore_barrier()
    pltpu.sync_copy(acc_sh.at[pl.ds(sid * 8192, 8192)],
                    out_hbm.at[pl.ds(cid * DEG_ACC + sid * 8192, 8192)])


def _make_prop(feature_split):
    @functools.partial(
        pl.kernel,
        out_type=jax.ShapeDtypeStruct((NC, NN, 16), jnp.float32),
        mesh=_sc_mesh(),
        compiler_params=pltpu.CompilerParams(use_tc_tiling_on_sc=False),
        scratch_types=[
            pltpu.VMEM((CHP, 128), jnp.int32),
            pltpu.VMEM((CHP, 128), jnp.int32),
            pltpu.VMEM((CHP, 128), jnp.int32),
            pltpu.VMEM((CHP, 128), jnp.int32),
            pltpu.VMEM((CHP, 128), jnp.int32),
            pltpu.VMEM((CHP, 128), jnp.int32),
            pltpu.VMEM((CHP, 128, 16), jnp.float32),
            pltpu.VMEM((CHP, 128, 16), jnp.float32),
            pltpu.VMEM((CHP, 128, 16), jnp.float32),
            pltpu.VMEM_SHARED((ACC_ROWS, 16), jnp.float32),
            pltpu.SemaphoreType.DMA,
            pltpu.SemaphoreType.DMA,
            pltpu.SemaphoreType.DMA,
            pltpu.SemaphoreType.DMA,
            pltpu.SemaphoreType.DMA,
        ])
    def prop(src_hbm, dst_hbm, t0_hbm, t1_hbm, zeros_hbm, out_hbm,
             sidx0, sidx1, sidx2, didx0, didx1, didx2,
             rows0, rows1, rows2, acc_sh,
             isem, gsem0, gsem1, gsem2, ssem):
        gsems = [gsem0, gsem1, gsem2]
        cid = lax.axis_index("c")
        sid = lax.axis_index("s")
        for q in range(3):
            pltpu.sync_copy(zeros_hbm,
                            acc_sh.at[pl.ds(sid * STRIPE + q * ZROWS, ZROWS)])
        pltpu.sync_copy(zeros_hbm.at[pl.ds(0, 1552)],
                        acc_sh.at[pl.ds(sid * STRIPE + 3 * ZROWS, 1552)])
        plsc.subcore_barrier()
        if feature_split:
            nrows, base = L1_RPS, sid * L1_RPS
        else:
            nrows, base = L2_RPW, (cid * NS + sid) * L2_RPW
        nt = nrows // CHP
        sbufs = [sidx0, sidx1, sidx2]
        dbufs = [didx0, didx1, didx2]
        rbufs = [rows0, rows1, rows2]

        def idx_cps(t, b):
            r0 = base + t * CHP
            return (pltpu.make_async_copy(src_hbm.at[pl.ds(r0, CHP)],
                                          sbufs[b], isem),
                    pltpu.make_async_copy(dst_hbm.at[pl.ds(r0, CHP)],
                                          dbufs[b], isem))

        def gath(tab, b):
            return [pltpu.async_copy(tab.at[sbufs[b].at[j]],
                                     rbufs[b].at[j], gsems[b])
                    for j in range(CHP)]

        def fire_gathers(b):
            if feature_split:
                @pl.when(cid == 0)
                def _():
                    gath(t0_hbm, b)

                @pl.when(cid == 1)
                def _():
                    gath(t1_hbm, b)
            else:
                gath(t0_hbm, b)

        def wait_gathers(b):
            for j in range(CHP):
                pltpu.make_async_copy(t0_hbm.at[pl.ds(0, 128)],
                                      rbufs[b].at[j], gsems[b]).wait()

        def drain_scatters(b):
            for j in range(CHP):
                pltpu.make_async_copy(t0_hbm.at[pl.ds(0, 128)],
                                      rbufs[b].at[j], ssem).wait()

        for cp in idx_cps(0, 0):
            cp.start()
        for cp in idx_cps(1, 1):
            cp.start()
        for cp in idx_cps(0, 0):
            cp.wait()
        fire_gathers(0)

        def group(g, c):
            for b in range(3):
                t = g * 3 + b
                bn, bp = (b + 1) % 3, (b + 2) % 3

                @pl.when(t + 1 < nt)
                def _():
                    for cp in idx_cps(t + 1, bn):
                        cp.wait()

                @pl.when(t > 0)
                def _():
                    drain_scatters(bp)

                @pl.when(t + 2 < nt)
                def _():
                    for cp in idx_cps(t + 2, bp):
                        cp.start()

                @pl.when(t + 1 < nt)
                def _():
                    fire_gathers(bn)

                wait_gathers(b)
                for j in range(CHP):
                    pltpu.async_copy(rbufs[b].at[j], acc_sh.at[dbufs[b].at[j]],
                                     ssem, add=True)
            return c

        lax.fori_loop(0, nt // 3, group, 0)
        drain_scatters((nt - 1) % 3)
        plsc.subcore_barrier()

        @pl.when(sid < NS - 1)
        def _():
            lo = sid * STRIPE
            pltpu.sync_copy(acc_sh.at[pl.ds(lo, STRIPE)],
                            out_hbm.at[cid, pl.ds(lo, STRIPE)])

        @pl.when(sid == NS - 1)
        def _():
            lo = (NS - 1) * STRIPE
            pltpu.sync_copy(acc_sh.at[pl.ds(lo, NN - lo)],
                            out_hbm.at[cid, pl.ds(lo, NN - lo)])

    return prop


_prop_fsplit = _make_prop(True)
_prop_esplit = _make_prop(False)


GW, RB = 10, 1250
_W3 = (GW, RB, 128)


def _stage_a_body(x_ref, w1a_ref, w1b_ref, dinv_ref, xs0_ref, xs1_ref,
                  xw0_ref, xw1_ref):
    x = x_ref[0]
    dinv = dinv_ref[0]
    xw0 = jnp.dot(x, w1a_ref[...], preferred_element_type=jnp.float32)
    xw1 = jnp.dot(x, w1b_ref[...], preferred_element_type=jnp.float32)
    xw0_ref[0] = xw0
    xw1_ref[0] = xw1
    xs0_ref[0] = xw0 * dinv
    xs1_ref[0] = xw1 * dinv


_BS = pl.BlockSpec((1, RB, 128), lambda i: (i, 0, 0))
_WS = pl.BlockSpec((128, 128), lambda i: (0, 0))
_VS = pl.BlockSpec((1, 128), lambda i: (0, 0))
_AS0 = pl.BlockSpec((1, 1, RB, 128), lambda i: (0, i, 0, 0))
_AS1 = pl.BlockSpec((1, 1, RB, 128), lambda i: (1, i, 0, 0))
_OW = jax.ShapeDtypeStruct(_W3, jnp.float32)


def _stage_a(xw_wide, w1a, w1b, dinvw):
    return pl.pallas_call(
        _stage_a_body,
        grid=(GW,),
        in_specs=[_BS, _WS, _WS, _BS],
        out_specs=[_BS, _BS, _BS, _BS],
        out_shape=[_OW, _OW, _OW, _OW],
    )(xw_wide, w1a, w1b, dinvw)


def _stage_b_body(a0_ref, a1_ref, xw0_ref, xw1_ref, dinv_ref, b1lo_ref,
                  b1hi_ref, w2a_ref, w2b_ref, hs_ref, hw_ref):
    dinv = dinv_ref[0]
    d2 = dinv * dinv
    h0 = jnp.maximum(a0_ref[0, 0] * dinv + xw0_ref[0] * d2 + b1lo_ref[...], 0.)
    h1 = jnp.maximum(a1_ref[0, 0] * dinv + xw1_ref[0] * d2 + b1hi_ref[...], 0.)
    hw = (jnp.dot(h0, w2a_ref[...], preferred_element_type=jnp.float32)
          + jnp.dot(h1, w2b_ref[...], preferred_element_type=jnp.float32))
    hw_ref[0] = hw
    hs_ref[0] = hw * dinv


def _stage_b(aggw, xw0, xw1, dinvw, b1lo, b1hi, w2a, w2b):
    return pl.pallas_call(
        _stage_b_body,
        grid=(GW,),
        in_specs=[_AS0, _AS1, _BS, _BS, _BS, _VS, _VS, _WS, _WS],
        out_specs=[_BS, _BS],
        out_shape=[_OW, _OW],
    )(aggw, aggw, xw0, xw1, dinvw, b1lo, b1hi, w2a, w2b)


def _stage_c_body(a0_ref, a1_ref, hw_ref, dinv_ref, b2_ref, out_ref):
    dinv = dinv_ref[0]
    agg = a0_ref[0, 0] + a1_ref[0, 0]
    out_ref[0] = dinv * agg + dinv * dinv * hw_ref[0] + b2_ref[...]


def _stage_c(agg2w, hw, dinvw, b2w):
    return pl.pallas_call(
        _stage_c_body,
        grid=(GW,),
        in_specs=[_AS0, _AS1, _BS, _BS, _VS],
        out_specs=_BS,
        out_shape=_OW,
    )(agg2w, agg2w, hw, dinvw, b2w)


def kernel(x, edge_index, W1, b1, W2, b2):
    src = edge_index[0].astype(jnp.int32)
    dst = edge_index[1].astype(jnp.int32)

    npad = EPAD - EE
    pad_i = jnp.arange(npad, dtype=jnp.int32)
    src_p = jnp.concatenate([src, pad_i % 128]).reshape(NROWS, 128)
    dst_p = jnp.concatenate([dst, NN + (pad_i % 32)]).reshape(NROWS, 128)

    ones128 = jnp.ones((128,), jnp.float32)
    zeros1d = jnp.zeros((8192,), jnp.float32)
    zeros2d = jnp.zeros((ZROWS, 16), jnp.float32)

    eye8 = jnp.eye(8, dtype=jnp.float32)
    w1a = jnp.kron(eye8, W1[:, :16])
    w1b = jnp.kron(eye8, W1[:, 16:])
    w2p = jnp.pad(W2, ((0, 0), (0, 14)))
    w2a = jnp.kron(eye8, w2p[:16, :])
    w2b = jnp.kron(eye8, w2p[16:, :])
    b1lo = jnp.tile(b1[:16], 8)[None, :]
    b1hi = jnp.tile(b1[16:], 8)[None, :]
    b2w = jnp.tile(jnp.pad(b2, (0, 14)), 8)[None, :]

    deg_flat = _degree_sc(dst_p, ones128, zeros1d)
    deg1d = deg_flat[:NN] + deg_flat[DEG_ACC:DEG_ACC + NN] + 1.0
    dinvw = jnp.broadcast_to(lax.rsqrt(deg1d)[:, None], (NN, 16)).reshape(_W3)

    x_wide = x.reshape(_W3)
    xs0w, xs1w, xw0w, xw1w = _stage_a(x_wide, w1a, w1b, dinvw)
    agg1 = _prop_fsplit(src_p, dst_p, xs0w.reshape(NN, 16),
                        xs1w.reshape(NN, 16), zeros2d)
    hsw, hww = _stage_b(agg1.reshape((NC,) + _W3), xw0w, xw1w, dinvw,
                        b1lo, b1hi, w2a, w2b)

    hs = hsw.reshape(NN, 16)
    agg2 = _prop_esplit(src_p, dst_p, hs, hs, zeros2d)
    out16w = _stage_c(agg2.reshape((NC,) + _W3), hww, dinvw, b2w)
    return out16w.reshape(NN, 16)[:, :2]
